# Initial kernel scaffold; baseline (speedup 1.0000x reference)
import jax
import jax.numpy as jnp
from jax.experimental import pallas as pl


def kernel(z, W):
    raise NotImplementedError("write your pallas kernel here")


# speedup vs baseline: 1.3745x; 1.3745x over previous
import functools

import jax
import jax.numpy as jnp
from jax import lax
from jax.experimental import pallas as pl
from jax.experimental.pallas import tpu as pltpu
from jax.experimental.pallas import tpu_sc as plsc

N_CODES = 8192
DIM = 256
N_TOKENS = 32768


_TM = 512


def _argmin_body(z_ref, w_ref, zss_ref, cbss_ref, out_ref):
    dot = lax.dot_general(
        z_ref[...], w_ref[...],
        dimension_numbers=(((1,), (1,)), ((), ())),
        preferred_element_type=jnp.float32,
    )
    dist = (zss_ref[...] - 2.0 * dot) + cbss_ref[...]
    dmin = jnp.min(dist, axis=1, keepdims=True)
    cols = lax.broadcasted_iota(jnp.int32, dist.shape, 1)
    idx = jnp.min(jnp.where(dist == dmin, cols, N_CODES), axis=1, keepdims=True)
    out_ref[...] = idx


def _run_argmin(zf, W, z_ss, cb_ss):
    nm = N_TOKENS // _TM
    return pl.pallas_call(
        _argmin_body,
        grid=(nm,),
        in_specs=[
            pl.BlockSpec((_TM, DIM), lambda i: (i, 0)),
            pl.BlockSpec((N_CODES, DIM), lambda i: (0, 0)),
            pl.BlockSpec((_TM, 1), lambda i: (i, 0)),
            pl.BlockSpec((1, N_CODES), lambda i: (0, 0)),
        ],
        out_specs=pl.BlockSpec((_TM, 1), lambda i: (i, 0)),
        out_shape=jax.ShapeDtypeStruct((N_TOKENS, 1), jnp.int32),
        compiler_params=pltpu.CompilerParams(
            dimension_semantics=("arbitrary",),
        ),
    )(zf, W, z_ss, cb_ss)


_BPW = 1024
_CH = 128
_NCHUNK = _BPW // _CH


def _sc_body(w_hbm, idx_hbm, zq_hbm, hist_hbm, idx_v, ones_v, buf0, buf1,
             zero_v, hist_sh, sem0, sem1):
    cid = lax.axis_index("c")
    sid = lax.axis_index("s")
    wid = cid * 16 + sid
    base = wid * _BPW

    pltpu.sync_copy(idx_hbm.at[wid], idx_v)

    for k in range(_CH // 16):
        ones_v[pl.ds(k * 16, 16)] = jnp.ones((16,), jnp.int32)

    @pl.when(sid == 0)
    def _zero_hist():
        def zbody(k, _):
            zero_v[pl.ds(k * 16, 16)] = jnp.zeros((16,), jnp.int32)
            return 0
        lax.fori_loop(0, N_CODES // 16, zbody, 0)
        pltpu.sync_copy(zero_v, hist_sh)

    plsc.subcore_barrier()
    for c in range(_NCHUNK):
        pltpu.sync_copy(ones_v, hist_sh.at[idx_v.at[c]], add=True)
    plsc.subcore_barrier()

    @pl.when(sid == 0)
    def _hist_out():
        pltpu.sync_copy(hist_sh, hist_hbm.at[cid])

    bufs = (buf0, buf1)
    sems = (sem0, sem1)
    cp = pltpu.async_copy(w_hbm.at[idx_v.at[0]], bufs[0], sems[0])
    for c in range(_NCHUNK):
        nxt = None
        if c + 1 < _NCHUNK:
            nxt = pltpu.async_copy(
                w_hbm.at[idx_v.at[c + 1]], bufs[(c + 1) % 2], sems[(c + 1) % 2])
        cp.wait()
        pltpu.sync_copy(bufs[c % 2], zq_hbm.at[pl.ds(base + c * _CH, _CH)])
        cp = nxt


def _run_sc(W, idx3):
    mesh = plsc.VectorSubcoreMesh(core_axis_name="c", subcore_axis_name="s")
    fn = functools.partial(
        pl.kernel,
        out_type=[
            jax.ShapeDtypeStruct((N_TOKENS, DIM), jnp.float32),
            jax.ShapeDtypeStruct((2, N_CODES), jnp.int32),
        ],
        scratch_types=[
            pltpu.VMEM((_NCHUNK, _CH), jnp.int32),
            pltpu.VMEM((_CH,), jnp.int32),
            pltpu.VMEM((_CH, DIM), jnp.float32),
            pltpu.VMEM((_CH, DIM), jnp.float32),
            pltpu.VMEM((N_CODES,), jnp.int32),
            pltpu.VMEM_SHARED((N_CODES,), jnp.int32),
            pltpu.SemaphoreType.DMA,
            pltpu.SemaphoreType.DMA,
        ],
        mesh=mesh,
    )(_sc_body)
    return fn(W, idx3)


_TM3 = 1024


def _loss_body(z_ref, q_ref, hist_ref, out_ref, commit_ref, ent_ref, acc_ref):
    m = pl.program_id(0)
    nm = pl.num_programs(0)
    zb = z_ref[...]
    qb = q_ref[...]
    diff = qb - zb
    out_ref[...] = zb + diff
    part = jnp.sum(diff * diff)

    @pl.when(m == 0)
    def _init():
        acc_ref[0, 0] = 0.0

    acc_ref[0, 0] += part

    @pl.when(m == nm - 1)
    def _finish():
        commit = acc_ref[0, 0] / jnp.float32(N_TOKENS * DIM)
        commit_ref[...] = jnp.reshape(commit, (1, 1))
        usage = (hist_ref[0, :] + hist_ref[1, :]).astype(jnp.float32)
        probs = usage / jnp.sum(usage)
        entropy = -jnp.sum(probs * jnp.log(probs + 1e-10))
        ent_ref[...] = jnp.reshape(jnp.log(jnp.float32(N_CODES)) - entropy,
                                   (1, 1))


def _run_loss(zf, zq, hist):
    nm = N_TOKENS // _TM3
    return pl.pallas_call(
        _loss_body,
        grid=(nm,),
        in_specs=[
            pl.BlockSpec((_TM3, DIM), lambda i: (i, 0)),
            pl.BlockSpec((_TM3, DIM), lambda i: (i, 0)),
            pl.BlockSpec((2, N_CODES), lambda i: (0, 0)),
        ],
        out_specs=[
            pl.BlockSpec((_TM3, DIM), lambda i: (i, 0)),
            pl.BlockSpec((1, 1), lambda i: (0, 0)),
            pl.BlockSpec((1, 1), lambda i: (0, 0)),
        ],
        out_shape=[
            jax.ShapeDtypeStruct((N_TOKENS, DIM), jnp.float32),
            jax.ShapeDtypeStruct((1, 1), jnp.float32),
            jax.ShapeDtypeStruct((1, 1), jnp.float32),
        ],
        scratch_shapes=[pltpu.SMEM((1, 1), jnp.float32)],
        compiler_params=pltpu.CompilerParams(
            dimension_semantics=("arbitrary",),
        ),
    )(zf, zq, hist)


def kernel(z, W):
    zf = z.reshape(N_TOKENS, DIM)
    z_ss = jnp.sum(zf ** 2, axis=1, keepdims=True)
    cb_ss = jnp.sum(W ** 2, axis=1).reshape(1, N_CODES)
    idx2d = _run_argmin(zf, W, z_ss, cb_ss)
    indices = idx2d.reshape(N_TOKENS)
    zq, hist = _run_sc(W, indices.reshape(32, _NCHUNK, _CH))
    zq_st, commit, ent = _run_loss(zf, zq, hist)
    return (zq_st.reshape(z.shape), indices, commit.reshape(()),
            ent.reshape(()))


# speedup vs baseline: 1.4912x; 1.0849x over previous
import functools

import jax
import jax.numpy as jnp
from jax import lax
from jax.experimental import pallas as pl
from jax.experimental.pallas import tpu as pltpu
from jax.experimental.pallas import tpu_sc as plsc

N_CODES = 8192
DIM = 256
N_TOKENS = 32768


_TM = 512


def _argmin_body(z_ref, w_ref, cbss_ref, cols_ref, out_ref):
    zb = z_ref[...]
    zss = jnp.sum(zb * zb, axis=1, keepdims=True)
    dot = lax.dot_general(
        zb * -2.0, w_ref[...],
        dimension_numbers=(((1,), (1,)), ((), ())),
        preferred_element_type=jnp.float32,
    )
    dist = (zss + dot) + cbss_ref[...]
    dmin = jnp.min(dist, axis=1, keepdims=True)
    cand = jnp.where(dist == dmin, cols_ref[...], jnp.float32(N_CODES))
    idx = jnp.min(cand, axis=1, keepdims=True)
    out_ref[...] = idx.astype(jnp.int32)


def _run_argmin(zf, W, cb_ss, cols):
    nm = N_TOKENS // _TM
    return pl.pallas_call(
        _argmin_body,
        grid=(nm,),
        in_specs=[
            pl.BlockSpec((_TM, DIM), lambda i: (i, 0)),
            pl.BlockSpec((N_CODES, DIM), lambda i: (0, 0)),
            pl.BlockSpec((1, N_CODES), lambda i: (0, 0)),
            pl.BlockSpec((1, N_CODES), lambda i: (0, 0)),
        ],
        out_specs=pl.BlockSpec((_TM, 1), lambda i: (i, 0)),
        out_shape=jax.ShapeDtypeStruct((N_TOKENS, 1), jnp.int32),
        compiler_params=pltpu.CompilerParams(
            dimension_semantics=("parallel",),
        ),
    )(zf, W, cb_ss, cols)


_BPW = 1024
_CH = 128
_NCHUNK = _BPW // _CH


def _sc_body(w_hbm, idx_hbm, zq_hbm, hist_hbm, idx_v, ones_v, buf0, buf1,
             zero_v, hist_sh, sem0, sem1):
    cid = lax.axis_index("c")
    sid = lax.axis_index("s")
    wid = cid * 16 + sid
    base = wid * _BPW

    pltpu.sync_copy(idx_hbm.at[wid], idx_v)

    for k in range(_CH // 16):
        ones_v[pl.ds(k * 16, 16)] = jnp.ones((16,), jnp.int32)

    @pl.when(sid == 0)
    def _zero_hist():
        def zbody(k, _):
            zero_v[pl.ds(k * 16, 16)] = jnp.zeros((16,), jnp.int32)
            return 0
        lax.fori_loop(0, N_CODES // 16, zbody, 0)
        pltpu.sync_copy(zero_v, hist_sh)

    plsc.subcore_barrier()
    for c in range(_NCHUNK):
        pltpu.sync_copy(ones_v, hist_sh.at[idx_v.at[c]], add=True)
    plsc.subcore_barrier()

    @pl.when(sid == 0)
    def _hist_out():
        pltpu.sync_copy(hist_sh, hist_hbm.at[cid])

    bufs = (buf0, buf1)
    sems = (sem0, sem1)
    cp = pltpu.async_copy(w_hbm.at[idx_v.at[0]], bufs[0], sems[0])
    for c in range(_NCHUNK):
        nxt = None
        if c + 1 < _NCHUNK:
            nxt = pltpu.async_copy(
                w_hbm.at[idx_v.at[c + 1]], bufs[(c + 1) % 2], sems[(c + 1) % 2])
        cp.wait()
        pltpu.sync_copy(bufs[c % 2], zq_hbm.at[pl.ds(base + c * _CH, _CH)])
        cp = nxt


def _run_sc(W, idx3):
    mesh = plsc.VectorSubcoreMesh(core_axis_name="c", subcore_axis_name="s")
    fn = functools.partial(
        pl.kernel,
        out_type=[
            jax.ShapeDtypeStruct((N_TOKENS, DIM), jnp.float32),
            jax.ShapeDtypeStruct((2, N_CODES), jnp.int32),
        ],
        scratch_types=[
            pltpu.VMEM((_NCHUNK, _CH), jnp.int32),
            pltpu.VMEM((_CH,), jnp.int32),
            pltpu.VMEM((_CH, DIM), jnp.float32),
            pltpu.VMEM((_CH, DIM), jnp.float32),
            pltpu.VMEM((N_CODES,), jnp.int32),
            pltpu.VMEM_SHARED((N_CODES,), jnp.int32),
            pltpu.SemaphoreType.DMA,
            pltpu.SemaphoreType.DMA,
        ],
        mesh=mesh,
    )(_sc_body)
    return fn(W, idx3)


_TM3 = 1024


def _loss_body(z_ref, q_ref, hist_ref, out_ref, commit_ref, ent_ref, acc_ref):
    m = pl.program_id(0)
    nm = pl.num_programs(0)
    zb = z_ref[...]
    qb = q_ref[...]
    diff = qb - zb
    out_ref[...] = zb + diff
    part = jnp.sum(diff * diff)

    @pl.when(m == 0)
    def _init():
        acc_ref[0, 0] = 0.0

    acc_ref[0, 0] += part

    @pl.when(m == nm - 1)
    def _finish():
        commit = acc_ref[0, 0] / jnp.float32(N_TOKENS * DIM)
        commit_ref[...] = jnp.reshape(commit, (1, 1))
        usage = (hist_ref[0, :] + hist_ref[1, :]).astype(jnp.float32)
        probs = usage / jnp.sum(usage)
        entropy = -jnp.sum(probs * jnp.log(probs + 1e-10))
        ent_ref[...] = jnp.reshape(jnp.log(jnp.float32(N_CODES)) - entropy,
                                   (1, 1))


def _run_loss(zf, zq, hist):
    nm = N_TOKENS // _TM3
    return pl.pallas_call(
        _loss_body,
        grid=(nm,),
        in_specs=[
            pl.BlockSpec((_TM3, DIM), lambda i: (i, 0)),
            pl.BlockSpec((_TM3, DIM), lambda i: (i, 0)),
            pl.BlockSpec((2, N_CODES), lambda i: (0, 0)),
        ],
        out_specs=[
            pl.BlockSpec((_TM3, DIM), lambda i: (i, 0)),
            pl.BlockSpec((1, 1), lambda i: (0, 0)),
            pl.BlockSpec((1, 1), lambda i: (0, 0)),
        ],
        out_shape=[
            jax.ShapeDtypeStruct((N_TOKENS, DIM), jnp.float32),
            jax.ShapeDtypeStruct((1, 1), jnp.float32),
            jax.ShapeDtypeStruct((1, 1), jnp.float32),
        ],
        scratch_shapes=[pltpu.SMEM((1, 1), jnp.float32)],
        compiler_params=pltpu.CompilerParams(
            dimension_semantics=("arbitrary",),
        ),
    )(zf, zq, hist)


def kernel(z, W):
    zf = z.reshape(N_TOKENS, DIM)
    cb_ss = jnp.sum(W ** 2, axis=1).reshape(1, N_CODES)
    cols = lax.broadcasted_iota(jnp.float32, (1, N_CODES), 1)
    idx2d = _run_argmin(zf, W, cb_ss, cols)
    indices = idx2d.reshape(N_TOKENS)
    zq, hist = _run_sc(W, indices.reshape(32, _NCHUNK, _CH))
    zq_st, commit, ent = _run_loss(zf, zq, hist)
    return (zq_st.reshape(z.shape), indices, commit.reshape(()),
            ent.reshape(()))


# speedup vs baseline: 1.5442x; 1.0355x over previous
import functools

import jax
import jax.numpy as jnp
from jax import lax
from jax.experimental import pallas as pl
from jax.experimental.pallas import tpu as pltpu
from jax.experimental.pallas import tpu_sc as plsc

N_CODES = 8192
DIM = 256
N_TOKENS = 32768


_TM = 1024


def _argmin_body(z_ref, w_ref, cbss_ref, cols_ref, out_ref):
    zb = z_ref[...]
    zss = jnp.sum(zb * zb, axis=1, keepdims=True)
    dot = lax.dot_general(
        zb * -2.0, w_ref[...],
        dimension_numbers=(((1,), (1,)), ((), ())),
        preferred_element_type=jnp.float32,
    )
    dist = (zss + dot) + cbss_ref[...]
    dmin = jnp.min(dist, axis=1, keepdims=True)
    cand = jnp.where(dist == dmin, cols_ref[...], jnp.float32(N_CODES))
    idx = jnp.min(cand, axis=1, keepdims=True)
    out_ref[...] = idx.astype(jnp.int32)


def _run_argmin(zf, W, cb_ss, cols):
    nm = N_TOKENS // _TM
    return pl.pallas_call(
        _argmin_body,
        grid=(nm,),
        in_specs=[
            pl.BlockSpec((_TM, DIM), lambda i: (i, 0)),
            pl.BlockSpec((N_CODES, DIM), lambda i: (0, 0)),
            pl.BlockSpec((1, N_CODES), lambda i: (0, 0)),
            pl.BlockSpec((1, N_CODES), lambda i: (0, 0)),
        ],
        out_specs=pl.BlockSpec((_TM, 1), lambda i: (i, 0)),
        out_shape=jax.ShapeDtypeStruct((N_TOKENS, 1), jnp.int32),
        compiler_params=pltpu.CompilerParams(
            dimension_semantics=("parallel",),
        ),
    )(zf, W, cb_ss, cols)


_BPW = 1024
_CH = 128
_NCHUNK = _BPW // _CH


def _sc_body(w_hbm, idx_hbm, zq_hbm, hist_hbm, idx_v, ones_v, buf0, buf1,
             zero_v, hist_sh, sem0, sem1):
    cid = lax.axis_index("c")
    sid = lax.axis_index("s")
    wid = cid * 16 + sid
    base = wid * _BPW

    pltpu.sync_copy(idx_hbm.at[wid], idx_v)

    for k in range(_CH // 16):
        ones_v[pl.ds(k * 16, 16)] = jnp.ones((16,), jnp.int32)

    @pl.when(sid == 0)
    def _zero_hist():
        def zbody(k, _):
            zero_v[pl.ds(k * 16, 16)] = jnp.zeros((16,), jnp.int32)
            return 0
        lax.fori_loop(0, N_CODES // 16, zbody, 0)
        pltpu.sync_copy(zero_v, hist_sh)

    plsc.subcore_barrier()
    for c in range(_NCHUNK):
        pltpu.sync_copy(ones_v, hist_sh.at[idx_v.at[c]], add=True)
    plsc.subcore_barrier()

    @pl.when(sid == 0)
    def _hist_out():
        pltpu.sync_copy(hist_sh, hist_hbm.at[cid])

    bufs = (buf0, buf1)
    sems = (sem0, sem1)
    cp = pltpu.async_copy(w_hbm.at[idx_v.at[0]], bufs[0], sems[0])
    for c in range(_NCHUNK):
        nxt = None
        if c + 1 < _NCHUNK:
            nxt = pltpu.async_copy(
                w_hbm.at[idx_v.at[c + 1]], bufs[(c + 1) % 2], sems[(c + 1) % 2])
        cp.wait()
        pltpu.sync_copy(bufs[c % 2], zq_hbm.at[pl.ds(base + c * _CH, _CH)])
        cp = nxt


def _run_sc(W, idx3):
    mesh = plsc.VectorSubcoreMesh(core_axis_name="c", subcore_axis_name="s")
    fn = functools.partial(
        pl.kernel,
        out_type=[
            jax.ShapeDtypeStruct((N_TOKENS, DIM), jnp.float32),
            jax.ShapeDtypeStruct((2, N_CODES), jnp.int32),
        ],
        scratch_types=[
            pltpu.VMEM((_NCHUNK, _CH), jnp.int32),
            pltpu.VMEM((_CH,), jnp.int32),
            pltpu.VMEM((_CH, DIM), jnp.float32),
            pltpu.VMEM((_CH, DIM), jnp.float32),
            pltpu.VMEM((N_CODES,), jnp.int32),
            pltpu.VMEM_SHARED((N_CODES,), jnp.int32),
            pltpu.SemaphoreType.DMA,
            pltpu.SemaphoreType.DMA,
        ],
        mesh=mesh,
    )(_sc_body)
    return fn(W, idx3)


_TM3 = 1024


def _loss_body(z_ref, q_ref, hist_ref, out_ref, commit_ref, ent_ref, acc_ref):
    m = pl.program_id(0)
    nm = pl.num_programs(0)
    zb = z_ref[...]
    qb = q_ref[...]
    diff = qb - zb
    out_ref[...] = zb + diff
    part = jnp.sum(diff * diff)

    @pl.when(m == 0)
    def _init():
        acc_ref[0, 0] = 0.0

    acc_ref[0, 0] += part

    @pl.when(m == nm - 1)
    def _finish():
        commit = acc_ref[0, 0] / jnp.float32(N_TOKENS * DIM)
        commit_ref[...] = jnp.reshape(commit, (1, 1))
        usage = (hist_ref[0, :] + hist_ref[1, :]).astype(jnp.float32)
        probs = usage / jnp.sum(usage)
        entropy = -jnp.sum(probs * jnp.log(probs + 1e-10))
        ent_ref[...] = jnp.reshape(jnp.log(jnp.float32(N_CODES)) - entropy,
                                   (1, 1))


def _run_loss(zf, zq, hist):
    nm = N_TOKENS // _TM3
    return pl.pallas_call(
        _loss_body,
        grid=(nm,),
        in_specs=[
            pl.BlockSpec((_TM3, DIM), lambda i: (i, 0)),
            pl.BlockSpec((_TM3, DIM), lambda i: (i, 0)),
            pl.BlockSpec((2, N_CODES), lambda i: (0, 0)),
        ],
        out_specs=[
            pl.BlockSpec((_TM3, DIM), lambda i: (i, 0)),
            pl.BlockSpec((1, 1), lambda i: (0, 0)),
            pl.BlockSpec((1, 1), lambda i: (0, 0)),
        ],
        out_shape=[
            jax.ShapeDtypeStruct((N_TOKENS, DIM), jnp.float32),
            jax.ShapeDtypeStruct((1, 1), jnp.float32),
            jax.ShapeDtypeStruct((1, 1), jnp.float32),
        ],
        scratch_shapes=[pltpu.SMEM((1, 1), jnp.float32)],
        compiler_params=pltpu.CompilerParams(
            dimension_semantics=("arbitrary",),
        ),
    )(zf, zq, hist)


def kernel(z, W):
    zf = z.reshape(N_TOKENS, DIM)
    cb_ss = jnp.sum(W ** 2, axis=1).reshape(1, N_CODES)
    cols = lax.broadcasted_iota(jnp.float32, (1, N_CODES), 1)
    idx2d = _run_argmin(zf, W, cb_ss, cols)
    indices = idx2d.reshape(N_TOKENS)
    zq, hist = _run_sc(W, indices.reshape(32, _NCHUNK, _CH))
    zq_st, commit, ent = _run_loss(zf, zq, hist)
    return (zq_st.reshape(z.shape), indices, commit.reshape(()),
            ent.reshape(()))


# speedup vs baseline: 1.6069x; 1.0406x over previous
import functools

import jax
import jax.numpy as jnp
from jax import lax
from jax.experimental import pallas as pl
from jax.experimental.pallas import tpu as pltpu
from jax.experimental.pallas import tpu_sc as plsc

N_CODES = 8192
DIM = 256
N_TOKENS = 32768
N_HALF = N_TOKENS // 2


_TM = 1024


def _argmin_body(z_ref, w_ref, cbss_ref, cols_ref, out_ref):
    zb = z_ref[...]
    zss = jnp.sum(zb * zb, axis=1, keepdims=True)
    dot = lax.dot_general(
        zb * -2.0, w_ref[...],
        dimension_numbers=(((1,), (1,)), ((), ())),
        preferred_element_type=jnp.float32,
    )
    dist = (zss + dot) + cbss_ref[...]
    dmin = jnp.min(dist, axis=1, keepdims=True)
    cand = jnp.where(dist == dmin, cols_ref[...], jnp.float32(N_CODES))
    idx = jnp.min(cand, axis=1, keepdims=True)
    out_ref[...] = idx.astype(jnp.int32)


def _run_argmin(zf, W, cb_ss, cols, half):
    nm = N_HALF // _TM
    off = half * nm
    return pl.pallas_call(
        _argmin_body,
        grid=(nm,),
        in_specs=[
            pl.BlockSpec((_TM, DIM), lambda i: (i + off, 0)),
            pl.BlockSpec((N_CODES, DIM), lambda i: (0, 0)),
            pl.BlockSpec((1, N_CODES), lambda i: (0, 0)),
            pl.BlockSpec((1, N_CODES), lambda i: (0, 0)),
        ],
        out_specs=pl.BlockSpec((_TM, 1), lambda i: (i, 0)),
        out_shape=jax.ShapeDtypeStruct((N_HALF, 1), jnp.int32),
        compiler_params=pltpu.CompilerParams(
            dimension_semantics=("parallel",),
        ),
    )(zf, W, cb_ss, cols)


_CH = 128
_BPW = N_HALF // 32
_NCHUNK = _BPW // _CH


def _sc_body(w_hbm, idx_hbm, zq_hbm, hist_hbm, idx_v, ones_v, buf0, buf1,
             zero_v, hist_sh, sem0, sem1):
    cid = lax.axis_index("c")
    sid = lax.axis_index("s")
    wid = cid * 16 + sid
    base = wid * _BPW

    pltpu.sync_copy(idx_hbm.at[wid], idx_v)

    for k in range(_CH // 16):
        ones_v[pl.ds(k * 16, 16)] = jnp.ones((16,), jnp.int32)

    @pl.when(sid == 0)
    def _zero_hist():
        def zbody(k, _):
            zero_v[pl.ds(k * 16, 16)] = jnp.zeros((16,), jnp.int32)
            return 0
        lax.fori_loop(0, N_CODES // 16, zbody, 0)
        pltpu.sync_copy(zero_v, hist_sh)

    plsc.subcore_barrier()
    for c in range(_NCHUNK):
        pltpu.sync_copy(ones_v, hist_sh.at[idx_v.at[c]], add=True)
    plsc.subcore_barrier()

    @pl.when(sid == 0)
    def _hist_out():
        pltpu.sync_copy(hist_sh, hist_hbm.at[cid])

    bufs = (buf0, buf1)
    sems = (sem0, sem1)
    cp = pltpu.async_copy(w_hbm.at[idx_v.at[0]], bufs[0], sems[0])
    for c in range(_NCHUNK):
        nxt = None
        if c + 1 < _NCHUNK:
            nxt = pltpu.async_copy(
                w_hbm.at[idx_v.at[c + 1]], bufs[(c + 1) % 2], sems[(c + 1) % 2])
        cp.wait()
        pltpu.sync_copy(bufs[c % 2], zq_hbm.at[pl.ds(base + c * _CH, _CH)])
        cp = nxt


def _run_sc(W, idx3):
    mesh = plsc.VectorSubcoreMesh(core_axis_name="c", subcore_axis_name="s")
    fn = functools.partial(
        pl.kernel,
        out_type=[
            jax.ShapeDtypeStruct((N_HALF, DIM), jnp.float32),
            jax.ShapeDtypeStruct((2, N_CODES), jnp.int32),
        ],
        scratch_types=[
            pltpu.VMEM((_NCHUNK, _CH), jnp.int32),
            pltpu.VMEM((_CH,), jnp.int32),
            pltpu.VMEM((_CH, DIM), jnp.float32),
            pltpu.VMEM((_CH, DIM), jnp.float32),
            pltpu.VMEM((N_CODES,), jnp.int32),
            pltpu.VMEM_SHARED((N_CODES,), jnp.int32),
            pltpu.SemaphoreType.DMA,
            pltpu.SemaphoreType.DMA,
        ],
        mesh=mesh,
    )(_sc_body)
    return fn(W, idx3)


_TM3 = 1024


def _loss_body(z_ref, qa_ref, qb_ref, hista_ref, histb_ref,
               out_ref, commit_ref, ent_ref, acc_ref):
    m = pl.program_id(0)
    nm = pl.num_programs(0)
    za = z_ref[0]
    zb = z_ref[1]
    qa = qa_ref[...]
    qb = qb_ref[...]
    da = qa - za
    db = qb - zb
    out_ref[0] = za + da
    out_ref[1] = zb + db
    part = jnp.sum(da * da) + jnp.sum(db * db)

    @pl.when(m == 0)
    def _init():
        acc_ref[0, 0] = 0.0

    acc_ref[0, 0] += part

    @pl.when(m == nm - 1)
    def _finish():
        commit = acc_ref[0, 0] / jnp.float32(N_TOKENS * DIM)
        commit_ref[...] = jnp.reshape(commit, (1, 1))
        usage = (hista_ref[0, :] + hista_ref[1, :]
                 + histb_ref[0, :] + histb_ref[1, :]).astype(jnp.float32)
        probs = usage / jnp.sum(usage)
        entropy = -jnp.sum(probs * jnp.log(probs + 1e-10))
        ent_ref[...] = jnp.reshape(jnp.log(jnp.float32(N_CODES)) - entropy,
                                   (1, 1))


def _run_loss(zf2, zqa, zqb, hista, histb):
    nm = N_HALF // _TM3
    return pl.pallas_call(
        _loss_body,
        grid=(nm,),
        in_specs=[
            pl.BlockSpec((2, _TM3, DIM), lambda i: (0, i, 0)),
            pl.BlockSpec((_TM3, DIM), lambda i: (i, 0)),
            pl.BlockSpec((_TM3, DIM), lambda i: (i, 0)),
            pl.BlockSpec((2, N_CODES), lambda i: (0, 0)),
            pl.BlockSpec((2, N_CODES), lambda i: (0, 0)),
        ],
        out_specs=[
            pl.BlockSpec((2, _TM3, DIM), lambda i: (0, i, 0)),
            pl.BlockSpec((1, 1), lambda i: (0, 0)),
            pl.BlockSpec((1, 1), lambda i: (0, 0)),
        ],
        out_shape=[
            jax.ShapeDtypeStruct((2, N_HALF, DIM), jnp.float32),
            jax.ShapeDtypeStruct((1, 1), jnp.float32),
            jax.ShapeDtypeStruct((1, 1), jnp.float32),
        ],
        scratch_shapes=[pltpu.SMEM((1, 1), jnp.float32)],
        compiler_params=pltpu.CompilerParams(
            dimension_semantics=("arbitrary",),
        ),
    )(zf2, zqa, zqb, hista, histb)


def kernel(z, W):
    zf = z.reshape(N_TOKENS, DIM)
    cb_ss = jnp.sum(W ** 2, axis=1).reshape(1, N_CODES)
    cols = lax.broadcasted_iota(jnp.float32, (1, N_CODES), 1)
    idxa = _run_argmin(zf, W, cb_ss, cols, 0)
    zqa, hista = _run_sc(W, idxa.reshape(32, _NCHUNK, _CH))
    idxb = _run_argmin(zf, W, cb_ss, cols, 1)
    zqb, histb = _run_sc(W, idxb.reshape(32, _NCHUNK, _CH))
    zf2 = zf.reshape(2, N_HALF, DIM)
    zq_st2, commit, ent = _run_loss(zf2, zqa, zqb, hista, histb)
    indices = jnp.concatenate(
        [idxa.reshape(N_HALF), idxb.reshape(N_HALF)])
    return (zq_st2.reshape(z.shape), indices, commit.reshape(()),
            ent.reshape(()))
